# baseline (speedup 1.0000x reference)
import jax
import jax.numpy as jnp
from jax.experimental import pallas as pl
from jax.experimental.pallas import tpu as pltpu

_EPS = 1e-4
_LANE = 128


def _stats_kernel(xr_ref, xi_ref, stats_ref):
    j = pl.program_id(1)
    xr = xr_ref[...]
    xi = xi_ref[...]
    tr = jnp.sum(xr, axis=0)
    ti = jnp.sum(xi, axis=0)
    srr = jnp.sum(xr * xr, axis=(0, 1))
    sii = jnp.sum(xi * xi, axis=(0, 1))
    sri = jnp.sum(xr * xi, axis=(0, 1))
    urr = jnp.sum(tr * tr, axis=0)
    uii = jnp.sum(ti * ti, axis=0)
    uri = jnp.sum(tr * ti, axis=0)
    z = jnp.zeros_like(srr)
    upd = jnp.stack([srr, sii, sri, urr, uii, uri, z, z], axis=0)[None]

    @pl.when(j == 0)
    def _():
        stats_ref[...] = upd

    @pl.when(j != 0)
    def _():
        stats_ref[...] += upd


def _make_apply_kernel(n_total, inv_b):
    inv_nm1 = 1.0 / (n_total - 1)

    def _apply_kernel(xr_ref, xi_ref, stats_ref, gr_ref, gi_ref, br_ref,
                      bi_ref, out_ref):
        stats = stats_ref[0] + stats_ref[1]
        srr, sii, sri = stats[0], stats[1], stats[2]
        urr, uii, uri = stats[3], stats[4], stats[5]
        a = (srr - urr * inv_b) * inv_nm1 + _EPS
        c = (sii - uii * inv_b) * inv_nm1 + _EPS
        b = (sri - uri * inv_b) * inv_nm1
        det = a * c - b * b
        s = jnp.sqrt(det)
        k = jax.lax.rsqrt(det * (a + c + 2.0 * s))
        w_rr = (c + s) * k
        w_ii = (a + s) * k
        w_ri = -b * k
        gr = gr_ref[0]
        gi = gi_ref[0]
        crr = gr * w_rr - gi * w_ri
        cri = gr * w_ri - gi * w_ii
        cir = gr * w_ri + gi * w_rr
        cii = gr * w_ii + gi * w_ri

        xr = xr_ref[...]
        xi = xi_ref[...]
        f = xr.shape[2] * xr.shape[3]
        mr = jnp.sum(xr, axis=(2, 3), keepdims=True) * (1.0 / f)
        mi = jnp.sum(xi, axis=(2, 3), keepdims=True) * (1.0 / f)
        xrc = xr - mr
        xic = xi - mi
        o_r = crr * xrc + cri * xic + br_ref[0]
        o_i = cir * xrc + cii * xic + bi_ref[0]

        lane = jax.lax.broadcasted_iota(jnp.int32, o_r.shape, 3)
        idx_lo = lane // 2
        idx_hi = idx_lo + _LANE // 2
        even = (lane % 2) == 0
        e_seg = jnp.where(even,
                          jnp.take_along_axis(o_r, idx_lo, axis=3),
                          jnp.take_along_axis(o_i, idx_lo, axis=3))
        o_seg = jnp.where(even,
                          jnp.take_along_axis(o_r, idx_hi, axis=3),
                          jnp.take_along_axis(o_i, idx_hi, axis=3))
        out_ref[...] = jnp.concatenate([e_seg, o_seg], axis=3)

    return _apply_kernel


def kernel(x_real, x_imag, gamma_r, gamma_i, beta_r, beta_i):
    B, C, F = x_real.shape
    NSEG = F // _LANE
    CC = 8
    NCORE = 2
    nc = C // CC
    half = nc // NCORE

    x_spec = pl.BlockSpec((B, CC, F), lambda i, j: (0, i * half + j, 0))
    stats = pl.pallas_call(
        _stats_kernel,
        grid=(NCORE, half),
        in_specs=[x_spec, x_spec],
        out_specs=pl.BlockSpec((1, 8, F), lambda i, j: (i, 0, 0)),
        out_shape=jax.ShapeDtypeStruct((NCORE, 8, F), jnp.float32),
        compiler_params=pltpu.CompilerParams(
            dimension_semantics=("parallel", "arbitrary"),
            vmem_limit_bytes=48 * 1024 * 1024,
        ),
        name="cln_stats",
    )(x_real, x_imag)

    BB = 4
    xb_spec = pl.BlockSpec((BB, C, NSEG, _LANE), lambda i: (i, 0, 0, 0))
    vec_spec = pl.BlockSpec((1, NSEG, _LANE), lambda i: (0, 0, 0))
    out = pl.pallas_call(
        _make_apply_kernel(B * C, 1.0 / B),
        grid=(B // BB,),
        in_specs=[
            xb_spec,
            xb_spec,
            pl.BlockSpec((NCORE, 8, NSEG, _LANE), lambda i: (0, 0, 0, 0)),
            vec_spec, vec_spec, vec_spec, vec_spec,
        ],
        out_specs=pl.BlockSpec((BB, C, NSEG, 2 * _LANE), lambda i: (i, 0, 0, 0)),
        out_shape=jax.ShapeDtypeStruct((B, C, NSEG, 2 * _LANE), jnp.float32),
        compiler_params=pltpu.CompilerParams(
            dimension_semantics=("parallel",),
            vmem_limit_bytes=56 * 1024 * 1024,
        ),
        name="cln_apply",
    )(x_real.reshape(B, C, NSEG, _LANE), x_imag.reshape(B, C, NSEG, _LANE),
      stats.reshape(NCORE, 8, NSEG, _LANE),
      gamma_r.reshape(1, NSEG, _LANE), gamma_i.reshape(1, NSEG, _LANE),
      beta_r.reshape(1, NSEG, _LANE), beta_i.reshape(1, NSEG, _LANE))

    return out.reshape(B, C, F, 2)


# speedup vs baseline: 3.6606x; 3.6606x over previous
import jax
import jax.numpy as jnp
from jax.experimental import pallas as pl
from jax.experimental.pallas import tpu as pltpu

_EPS = 1e-4


def _stats_kernel(xr_ref, xi_ref, stats_ref):
    j = pl.program_id(0)
    xr = xr_ref[...]
    xi = xi_ref[...]
    tr = jnp.sum(xr, axis=0)
    ti = jnp.sum(xi, axis=0)
    srr = jnp.sum(xr * xr, axis=(0, 1))
    sii = jnp.sum(xi * xi, axis=(0, 1))
    sri = jnp.sum(xr * xi, axis=(0, 1))
    urr = jnp.sum(tr * tr, axis=0)
    uii = jnp.sum(ti * ti, axis=0)
    uri = jnp.sum(tr * ti, axis=0)
    z = jnp.zeros_like(srr)
    upd = jnp.stack([srr, sii, sri, urr, uii, uri, z, z], axis=0)

    @pl.when(j == 0)
    def _():
        stats_ref[...] = upd

    @pl.when(j != 0)
    def _():
        stats_ref[...] += upd


def _make_apply_kernel(n_total, inv_b):
    inv_nm1 = 1.0 / (n_total - 1)

    def _apply_kernel(xr_ref, xi_ref, stats_ref, gr_ref, gi_ref, br_ref,
                      bi_ref, out_ref):
        stats = stats_ref[...]
        srr, sii, sri = stats[0], stats[1], stats[2]
        urr, uii, uri = stats[3], stats[4], stats[5]
        a = (srr - urr * inv_b) * inv_nm1 + _EPS
        c = (sii - uii * inv_b) * inv_nm1 + _EPS
        b = (sri - uri * inv_b) * inv_nm1
        det = a * c - b * b
        s = jnp.sqrt(det)
        k = jax.lax.rsqrt(det * (a + c + 2.0 * s))
        w_rr = (c + s) * k
        w_ii = (a + s) * k
        w_ri = -b * k
        gr = gr_ref[0]
        gi = gi_ref[0]
        crr = gr * w_rr - gi * w_ri
        cri = gr * w_ri - gi * w_ii
        cir = gr * w_ri + gi * w_rr
        cii = gr * w_ii + gi * w_ri

        xr = xr_ref[...]
        xi = xi_ref[...]
        f = xr.shape[-1]
        mr = jnp.sum(xr, axis=2, keepdims=True) * (1.0 / f)
        mi = jnp.sum(xi, axis=2, keepdims=True) * (1.0 / f)
        xrc = xr - mr
        xic = xi - mi
        out_ref[:, :, 0, :] = crr * xrc + cri * xic + br_ref[0]
        out_ref[:, :, 1, :] = cir * xrc + cii * xic + bi_ref[0]

    return _apply_kernel


def kernel(x_real, x_imag, gamma_r, gamma_i, beta_r, beta_i):
    B, C, F = x_real.shape
    CC = 16
    nc = C // CC

    x_spec = pl.BlockSpec((B, CC, F), lambda j: (0, j, 0))
    stats = pl.pallas_call(
        _stats_kernel,
        grid=(nc,),
        in_specs=[x_spec, x_spec],
        out_specs=pl.BlockSpec((8, F), lambda j: (0, 0)),
        out_shape=jax.ShapeDtypeStruct((8, F), jnp.float32),
        compiler_params=pltpu.CompilerParams(
            dimension_semantics=("arbitrary",),
            vmem_limit_bytes=48 * 1024 * 1024,
        ),
        name="cln_stats",
    )(x_real, x_imag)

    BB = 4
    xb_spec = pl.BlockSpec((BB, C, F), lambda i: (i, 0, 0))
    vec_spec = pl.BlockSpec((1, F), lambda i: (0, 0))
    out = pl.pallas_call(
        _make_apply_kernel(B * C, 1.0 / B),
        grid=(B // BB,),
        in_specs=[
            xb_spec,
            xb_spec,
            pl.BlockSpec((8, F), lambda i: (0, 0)),
            vec_spec, vec_spec, vec_spec, vec_spec,
        ],
        out_specs=pl.BlockSpec((BB, C, 2, F), lambda i: (i, 0, 0, 0)),
        out_shape=jax.ShapeDtypeStruct((B, C, 2, F), jnp.float32),
        compiler_params=pltpu.CompilerParams(
            dimension_semantics=("parallel",),
            vmem_limit_bytes=56 * 1024 * 1024,
        ),
        name="cln_apply",
    )(x_real, x_imag, stats,
      gamma_r.reshape(1, F), gamma_i.reshape(1, F),
      beta_r.reshape(1, F), beta_i.reshape(1, F))

    return out.swapaxes(2, 3)


# speedup vs baseline: 3.6802x; 1.0054x over previous
import jax
import jax.numpy as jnp
from jax.experimental import pallas as pl
from jax.experimental.pallas import tpu as pltpu

_EPS = 1e-4


def _make_fused_kernel(n_total, inv_b):
    inv_nm1 = 1.0 / (n_total - 1)

    def _fused_kernel(xr_ref, xi_ref, gr_ref, gi_ref, br_ref, bi_ref,
                      out_ref, acc_ref, coef_ref):
        p = pl.program_id(0)
        j = pl.program_id(1)

        @pl.when(p == 0)
        def _stats_phase():
            xr = xr_ref[...]
            xi = xi_ref[...]
            tr = jnp.sum(xr, axis=0)
            ti = jnp.sum(xi, axis=0)
            srr = jnp.sum(xr * xr, axis=(0, 1))
            sii = jnp.sum(xi * xi, axis=(0, 1))
            sri = jnp.sum(xr * xi, axis=(0, 1))
            urr = jnp.sum(tr * tr, axis=0)
            uii = jnp.sum(ti * ti, axis=0)
            uri = jnp.sum(tr * ti, axis=0)
            z = jnp.zeros_like(srr)
            upd = jnp.stack([srr, sii, sri, urr, uii, uri, z, z], axis=0)

            @pl.when(j == 0)
            def _():
                acc_ref[...] = upd

            @pl.when(j != 0)
            def _():
                acc_ref[...] += upd

        @pl.when((p == 1) & (j == 0))
        def _coef_phase():
            stats = acc_ref[...]
            srr, sii, sri = stats[0], stats[1], stats[2]
            urr, uii, uri = stats[3], stats[4], stats[5]
            a = (srr - urr * inv_b) * inv_nm1 + _EPS
            c = (sii - uii * inv_b) * inv_nm1 + _EPS
            b = (sri - uri * inv_b) * inv_nm1
            det = a * c - b * b
            s = jnp.sqrt(det)
            k = jax.lax.rsqrt(det * (a + c + 2.0 * s))
            w_rr = (c + s) * k
            w_ii = (a + s) * k
            w_ri = -b * k
            gr = gr_ref[0]
            gi = gi_ref[0]
            coef_ref[0, :] = gr * w_rr - gi * w_ri
            coef_ref[1, :] = gr * w_ri - gi * w_ii
            coef_ref[2, :] = gr * w_ri + gi * w_rr
            coef_ref[3, :] = gr * w_ii + gi * w_ri

        @pl.when(p == 1)
        def _apply_phase():
            crr = coef_ref[0, :]
            cri = coef_ref[1, :]
            cir = coef_ref[2, :]
            cii = coef_ref[3, :]
            xr = xr_ref[...]
            xi = xi_ref[...]
            f = xr.shape[-1]
            mr = jnp.sum(xr, axis=2, keepdims=True) * (1.0 / f)
            mi = jnp.sum(xi, axis=2, keepdims=True) * (1.0 / f)
            xrc = xr - mr
            xic = xi - mi
            out_ref[:, :, 0, :] = crr * xrc + cri * xic + br_ref[0]
            out_ref[:, :, 1, :] = cir * xrc + cii * xic + bi_ref[0]

    return _fused_kernel


def kernel(x_real, x_imag, gamma_r, gamma_i, beta_r, beta_i):
    B, C, F = x_real.shape
    CC = 8
    nc = C // CC

    x_spec = pl.BlockSpec((B, CC, F), lambda p, j: (0, j, 0))
    vec_spec = pl.BlockSpec((1, F), lambda p, j: (0, 0))
    out = pl.pallas_call(
        _make_fused_kernel(B * C, 1.0 / B),
        grid=(2, nc),
        in_specs=[x_spec, x_spec, vec_spec, vec_spec, vec_spec, vec_spec],
        out_specs=pl.BlockSpec(
            (B, CC, 2, F),
            lambda p, j: (0, jnp.where(p == 1, j, 0), 0, 0)),
        out_shape=jax.ShapeDtypeStruct((B, C, 2, F), jnp.float32),
        scratch_shapes=[
            pltpu.VMEM((8, F), jnp.float32),
            pltpu.VMEM((4, F), jnp.float32),
        ],
        compiler_params=pltpu.CompilerParams(
            dimension_semantics=("arbitrary", "arbitrary"),
            vmem_limit_bytes=56 * 1024 * 1024,
        ),
        name="cln_fused",
    )(x_real, x_imag,
      gamma_r.reshape(1, F), gamma_i.reshape(1, F),
      beta_r.reshape(1, F), beta_i.reshape(1, F))

    return out.swapaxes(2, 3)


# speedup vs baseline: 3.6907x; 1.0029x over previous
import jax
import jax.numpy as jnp
from jax.experimental import pallas as pl
from jax.experimental.pallas import tpu as pltpu

_EPS = 1e-4
_NSTASH = 1


def _make_fused_kernel(n_total, inv_b):
    inv_nm1 = 1.0 / (n_total - 1)

    def _fused_kernel(xr_ref, xi_ref, gr_ref, gi_ref, br_ref, bi_ref,
                      out_ref, acc_ref, coef_ref, stash_r, stash_i):
        p = pl.program_id(0)
        j = pl.program_id(1)

        @pl.when(p == 0)
        def _stats_phase():
            xr = xr_ref[...]
            xi = xi_ref[...]
            tr = jnp.sum(xr, axis=0)
            ti = jnp.sum(xi, axis=0)
            srr = jnp.sum(xr * xr, axis=(0, 1))
            sii = jnp.sum(xi * xi, axis=(0, 1))
            sri = jnp.sum(xr * xi, axis=(0, 1))
            urr = jnp.sum(tr * tr, axis=0)
            uii = jnp.sum(ti * ti, axis=0)
            uri = jnp.sum(tr * ti, axis=0)
            z = jnp.zeros_like(srr)
            upd = jnp.stack([srr, sii, sri, urr, uii, uri, z, z], axis=0)

            @pl.when(j == 0)
            def _():
                acc_ref[...] = upd

            @pl.when(j != 0)
            def _():
                acc_ref[...] += upd

            for sj in range(_NSTASH):
                @pl.when(j == sj)
                def _(sj=sj):
                    stash_r[sj] = xr_ref[...]
                    stash_i[sj] = xi_ref[...]

        @pl.when((p == 1) & (j == 0))
        def _coef_phase():
            stats = acc_ref[...]
            srr, sii, sri = stats[0], stats[1], stats[2]
            urr, uii, uri = stats[3], stats[4], stats[5]
            a = (srr - urr * inv_b) * inv_nm1 + _EPS
            c = (sii - uii * inv_b) * inv_nm1 + _EPS
            b = (sri - uri * inv_b) * inv_nm1
            det = a * c - b * b
            s = jnp.sqrt(det)
            k = jax.lax.rsqrt(det * (a + c + 2.0 * s))
            w_rr = (c + s) * k
            w_ii = (a + s) * k
            w_ri = -b * k
            gr = gr_ref[0]
            gi = gi_ref[0]
            coef_ref[0, :] = gr * w_rr - gi * w_ri
            coef_ref[1, :] = gr * w_ri - gi * w_ii
            coef_ref[2, :] = gr * w_ri + gi * w_rr
            coef_ref[3, :] = gr * w_ii + gi * w_ri

        def _do_apply(xr, xi):
            crr = coef_ref[0, :]
            cri = coef_ref[1, :]
            cir = coef_ref[2, :]
            cii = coef_ref[3, :]
            f = xr.shape[-1]
            mr = jnp.sum(xr, axis=2, keepdims=True) * (1.0 / f)
            mi = jnp.sum(xi, axis=2, keepdims=True) * (1.0 / f)
            xrc = xr - mr
            xic = xi - mi
            out_ref[:, :, 0, :] = crr * xrc + cri * xic + br_ref[0]
            out_ref[:, :, 1, :] = cir * xrc + cii * xic + bi_ref[0]

        for sj in range(_NSTASH):
            @pl.when((p == 1) & (j == sj))
            def _(sj=sj):
                _do_apply(stash_r[sj], stash_i[sj])

        @pl.when((p == 1) & (j >= _NSTASH))
        def _apply_phase():
            _do_apply(xr_ref[...], xi_ref[...])

    return _fused_kernel


def kernel(x_real, x_imag, gamma_r, gamma_i, beta_r, beta_i):
    B, C, F = x_real.shape
    CC = 8
    nc = C // CC

    x_spec = pl.BlockSpec(
        (B, CC, F),
        lambda p, j: (0, jnp.where((p == 1) & (j < _NSTASH), nc - 1, j), 0))
    vec_spec = pl.BlockSpec((1, F), lambda p, j: (0, 0))
    out = pl.pallas_call(
        _make_fused_kernel(B * C, 1.0 / B),
        grid=(2, nc),
        in_specs=[x_spec, x_spec, vec_spec, vec_spec, vec_spec, vec_spec],
        out_specs=pl.BlockSpec(
            (B, CC, 2, F),
            lambda p, j: (0, jnp.where(p == 1, j, 0), 0, 0)),
        out_shape=jax.ShapeDtypeStruct((B, C, 2, F), jnp.float32),
        scratch_shapes=[
            pltpu.VMEM((8, F), jnp.float32),
            pltpu.VMEM((4, F), jnp.float32),
            pltpu.VMEM((_NSTASH, B, CC, F), jnp.float32),
            pltpu.VMEM((_NSTASH, B, CC, F), jnp.float32),
        ],
        compiler_params=pltpu.CompilerParams(
            dimension_semantics=("arbitrary", "arbitrary"),
            vmem_limit_bytes=56 * 1024 * 1024,
        ),
        name="cln_fused",
    )(x_real, x_imag,
      gamma_r.reshape(1, F), gamma_i.reshape(1, F),
      beta_r.reshape(1, F), beta_i.reshape(1, F))

    return out.swapaxes(2, 3)
